# baseline (speedup 1.0000x reference)
import functools

import jax
import jax.numpy as jnp
from jax import lax
from jax.experimental import pallas as pl
from jax.experimental.pallas import tpu as pltpu
from jax.experimental.pallas import tpu_sc as plsc

CHUNK = 800


@functools.lru_cache(maxsize=None)
def _build(batch: int, seq: int, vocab: int, dim: int):
    info = plsc.get_sparse_core_info()
    nw = info.num_cores * info.num_subcores
    bw = batch // nw
    cb = CHUNK // seq
    assert batch % nw == 0 and CHUNK % seq == 0 and bw % cb == 0
    n_chunks = bw // cb
    assert n_chunks % 2 == 0
    n2 = n_chunks // 2

    mesh = plsc.VectorSubcoreMesh(core_axis_name="c", subcore_axis_name="s")

    @functools.partial(
        pl.kernel,
        mesh=mesh,
        out_type=jax.ShapeDtypeStruct((batch, seq, dim), jnp.float32),
        scratch_types=[
            pltpu.VMEM((bw * seq,), jnp.int32),
            pltpu.VMEM((2, cb * seq, dim), jnp.float32),
            pltpu.SemaphoreType.DMA,
            pltpu.SemaphoreType.DMA,
            pltpu.SemaphoreType.DMA,
            pltpu.SemaphoreType.DMA,
        ],
        compiler_params=pltpu.CompilerParams(use_tc_tiling_on_sc=False),
    )
    def gather_kernel(x_hbm, table_hbm, out_hbm, idx_v, rows_v, sg0, sg1, so0, so1):
        wid = lax.axis_index("s") * info.num_cores + lax.axis_index("c")
        base = wid * bw
        sg = (sg0, sg1)
        so = (so0, so1)

        pltpu.sync_copy(x_hbm.at[pl.ds(base * seq, bw * seq)], idx_v)

        def gat(i, b):
            return pltpu.make_async_copy(
                table_hbm.at[idx_v.at[pl.ds(i * cb * seq, cb * seq)]],
                rows_v.at[b],
                sg[b],
            )

        def sto_ops(i, b):
            return [
                pltpu.make_async_copy(
                    rows_v.at[b, pl.ds(r * seq, seq)],
                    out_hbm.at[base + i * cb + r],
                    so[b],
                )
                for r in range(cb)
            ]

        class _Sto:
            def __init__(self, i, b):
                self.i, self.b = i, b

            def start(self):
                for op in sto_ops(self.i, self.b):
                    op.start()

            def wait(self):
                for op in sto_ops(self.i, self.b):
                    op.wait()

        def sto(i, b):
            return _Sto(i, b)

        gat(0, 0).start()

        def body(j, carry):
            i0 = 2 * j
            i1 = i0 + 1
            gat(i0, 0).wait()
            gat(i1, 1).start()
            sto(i0, 0).start()

            @pl.when(j > 0)
            def _():
                sto(i0 - 1, 1).wait()

            gat(i1, 1).wait()
            sto(i1, 1).start()

            @pl.when(j < n2 - 1)
            def _():
                sto(i0, 0).wait()
                gat(i0 + 2, 0).start()

            return carry

        lax.fori_loop(0, n2, body, 0)
        sto(n_chunks - 2, 0).wait()
        sto(n_chunks - 1, 1).wait()

    return gather_kernel


def kernel(x, table):
    batch, seq = x.shape
    vocab, dim = table.shape
    fn = _build(batch, seq, vocab, dim)
    return fn(x.reshape(-1).astype(jnp.int32), table)
